# Initial kernel scaffold; baseline (speedup 1.0000x reference)
import jax
import jax.numpy as jnp
from jax.experimental import pallas as pl


def kernel(x, edge_index, edge_weight, weight, bias, attn_w, W1, b1, W2, b2, bn1_g, bn1_b, bn2_g, bn2_b, lin_W, lin_b):
    raise NotImplementedError("write your pallas kernel here")


# speedup vs baseline: 5.7209x; 5.7209x over previous
import functools

import jax
import jax.numpy as jnp
from jax import lax
from jax.experimental import pallas as pl
from jax.experimental.pallas import tpu as pltpu
from jax.experimental.pallas import tpu_sc as plsc

_N = 52
_E = 832
_WIN = 5
_OUT = 128
_LANES = 16
_EG = _E // _LANES


def _sc_edge_scatter_body(zeros_hbm, src_hbm, dst_hbm, ew_hbm, out_hbm,
                          acc_v, src_v, dst_v, ew_v):
    cid = lax.axis_index("c")
    sid = lax.axis_index("s")

    @pl.when(jnp.logical_and(cid == 0, sid == 0))
    def _():
        pltpu.sync_copy(zeros_hbm, acc_v)
        pltpu.sync_copy(src_hbm, src_v)
        pltpu.sync_copy(dst_hbm, dst_v)
        pltpu.sync_copy(ew_hbm, ew_v)
        lane_base = lax.iota(jnp.int32, _LANES) * (_N * _N)

        def body(g, carry):
            off = pl.multiple_of(g * _LANES, _LANES)
            s = src_v[pl.ds(off, _LANES)]
            d = dst_v[pl.ds(off, _LANES)]
            w = ew_v[pl.ds(off, _LANES)]
            plsc.addupdate_scatter(acc_v, [lane_base + d * _N + s], w)
            return carry

        lax.fori_loop(0, _EG, body, 0)
        pltpu.sync_copy(acc_v, out_hbm)


@functools.cache
def _sc_edge_scatter():
    return pl.kernel(
        _sc_edge_scatter_body,
        out_type=jax.ShapeDtypeStruct((_LANES * _N * _N,), jnp.float32),
        mesh=plsc.VectorSubcoreMesh(core_axis_name="c", subcore_axis_name="s"),
        compiler_params=pltpu.CompilerParams(needs_layout_passes=False),
        scratch_types=[
            pltpu.VMEM((_LANES * _N * _N,), jnp.float32),
            pltpu.VMEM((_E,), jnp.int32),
            pltpu.VMEM((_E,), jnp.int32),
            pltpu.VMEM((_E,), jnp.float32),
        ],
    )


def _tc_body(x_ref, w_ref, b_ref, aw_ref, W1_ref, b1_ref, W2_ref, b2_ref,
             g1_ref, be1_ref, g2_ref, be2_ref, lw_ref, lb_ref, acc_ref,
             o_ref):
    hs = [jnp.dot(x_ref[t], w_ref[t], preferred_element_type=jnp.float32)
          for t in range(_WIN)]
    att = aw_ref[...]
    ss = [jnp.sum(h * att, axis=1, keepdims=True) for h in hs]
    m = ss[0]
    for s in ss[1:]:
        m = jnp.maximum(m, s)
    es = [jnp.exp(s - m) for s in ss]
    z = es[0]
    for e in es[1:]:
        z = z + e
    h = es[0] * hs[0]
    for t in range(1, _WIN):
        h = h + es[t] * hs[t]
    h = h / z + b_ref[...]

    a_raw = acc_ref[0]
    for c in range(1, _LANES):
        a_raw = a_raw + acc_ref[c]
    deg = jnp.sum(a_raw, axis=1, keepdims=True) + 1.0
    dinv = lax.rsqrt(deg)
    rr = lax.broadcasted_iota(jnp.int32, (_N, _N), 0)
    cc = lax.broadcasted_iota(jnp.int32, (_N, _N), 1)
    a_n = jnp.where(rr == cc, a_raw + 1.0, a_raw)

    def gcn(hin, W_r, bb_r):
        hw = jnp.dot(hin, W_r[...], preferred_element_type=jnp.float32)
        agg = jnp.dot(a_n, dinv * hw, preferred_element_type=jnp.float32)
        return dinv * agg + bb_r[...]

    def norm_relu(v, g_r, be_r):
        mean = jnp.mean(v, axis=1, keepdims=True)
        cen = v - mean
        var = jnp.mean(cen * cen, axis=1, keepdims=True)
        vn = cen * lax.rsqrt(var + 1e-5) * g_r[...] + be_r[...]
        return jnp.maximum(vn, 0.0)

    h1 = norm_relu(gcn(h, W1_ref, b1_ref), g1_ref, be1_ref)
    h2 = norm_relu(gcn(h1, W2_ref, b2_ref), g2_ref, be2_ref)
    o_ref[...] = (jnp.dot(h2, lw_ref[...], preferred_element_type=jnp.float32)
                  + lb_ref[...])


def _tc_call(args, interpret=False):
    return pl.pallas_call(
        _tc_body,
        out_shape=jax.ShapeDtypeStruct((_N, _OUT), jnp.float32),
        interpret=interpret,
    )(*args)


def kernel(x, edge_index, edge_weight, weight, bias, attn_w, W1, b1, W2, b2,
           bn1_g, bn1_b, bn2_g, bn2_b, lin_W, lin_b):
    ei = jnp.asarray(edge_index, jnp.int32)
    zeros = jnp.zeros((_LANES * _N * _N,), jnp.float32)
    acc = _sc_edge_scatter()(zeros, ei[0], ei[1],
                             jnp.asarray(edge_weight, jnp.float32))
    acc = acc.reshape(_LANES, _N, _N)
    return _tc_call((
        x, weight,
        bias.reshape(1, -1), attn_w.reshape(1, -1),
        W1, b1.reshape(1, -1), W2, b2.reshape(1, -1),
        bn1_g.reshape(-1, 1), bn1_b.reshape(-1, 1),
        bn2_g.reshape(-1, 1), bn2_b.reshape(-1, 1),
        lin_W, lin_b.reshape(1, -1), acc,
    ))


# speedup vs baseline: 6.3487x; 1.1097x over previous
import functools

import jax
import jax.numpy as jnp
from jax import lax
from jax.experimental import pallas as pl
from jax.experimental.pallas import tpu as pltpu
from jax.experimental.pallas import tpu_sc as plsc

_N = 52
_E = 832
_WIN = 5
_OUT = 128
_LANES = 16
_EG = _E // _LANES


def _sc_edge_scatter_body(zeros_hbm, src_hbm, dst_hbm, ew_hbm, out_hbm,
                          acc_v, src_v, dst_v, ew_v):
    cid = lax.axis_index("c")
    sid = lax.axis_index("s")

    @pl.when(jnp.logical_and(cid == 0, sid == 0))
    def _():
        pltpu.sync_copy(zeros_hbm, acc_v)
        pltpu.sync_copy(src_hbm, src_v)
        pltpu.sync_copy(dst_hbm, dst_v)
        pltpu.sync_copy(ew_hbm, ew_v)

        def body(g, carry):
            off = pl.multiple_of(g * _LANES, _LANES)
            s = src_v[pl.ds(off, _LANES)]
            d = dst_v[pl.ds(off, _LANES)]
            w = ew_v[pl.ds(off, _LANES)]
            plsc.addupdate_scatter(acc_v, [d * _N + s], w)
            return carry

        lax.fori_loop(0, _EG, body, 0)
        pltpu.sync_copy(acc_v, out_hbm)


@functools.cache
def _sc_edge_scatter():
    return pl.kernel(
        _sc_edge_scatter_body,
        out_type=jax.ShapeDtypeStruct((_N * _N,), jnp.float32),
        mesh=plsc.VectorSubcoreMesh(core_axis_name="c", subcore_axis_name="s"),
        compiler_params=pltpu.CompilerParams(needs_layout_passes=False),
        scratch_types=[
            pltpu.VMEM((_N * _N,), jnp.float32),
            pltpu.VMEM((_E,), jnp.int32),
            pltpu.VMEM((_E,), jnp.int32),
            pltpu.VMEM((_E,), jnp.float32),
        ],
    )


def _tc_body(x_ref, w_ref, b_ref, aw_ref, W1_ref, b1_ref, W2_ref, b2_ref,
             g1_ref, be1_ref, g2_ref, be2_ref, lw_ref, lb_ref, acc_ref,
             o_ref):
    hs = [jnp.dot(x_ref[t], w_ref[t], preferred_element_type=jnp.float32)
          for t in range(_WIN)]
    att = aw_ref[...]
    ss = [jnp.sum(h * att, axis=1, keepdims=True) for h in hs]
    m = ss[0]
    for s in ss[1:]:
        m = jnp.maximum(m, s)
    es = [jnp.exp(s - m) for s in ss]
    z = es[0]
    for e in es[1:]:
        z = z + e
    h = es[0] * hs[0]
    for t in range(1, _WIN):
        h = h + es[t] * hs[t]
    h = h / z + b_ref[...]

    a_raw = acc_ref[...]
    deg = jnp.sum(a_raw, axis=1, keepdims=True) + 1.0
    dinv = lax.rsqrt(deg)
    rr = lax.broadcasted_iota(jnp.int32, (_N, _N), 0)
    cc = lax.broadcasted_iota(jnp.int32, (_N, _N), 1)
    a_n = jnp.where(rr == cc, a_raw + 1.0, a_raw)

    def gcn(hin, W_r, bb_r):
        hw = jnp.dot(hin, W_r[...], preferred_element_type=jnp.float32)
        agg = jnp.dot(a_n, dinv * hw, preferred_element_type=jnp.float32)
        return dinv * agg + bb_r[...]

    def norm_relu(v, g_r, be_r):
        mean = jnp.mean(v, axis=1, keepdims=True)
        cen = v - mean
        var = jnp.mean(cen * cen, axis=1, keepdims=True)
        vn = cen * lax.rsqrt(var + 1e-5) * g_r[...] + be_r[...]
        return jnp.maximum(vn, 0.0)

    h1 = norm_relu(gcn(h, W1_ref, b1_ref), g1_ref, be1_ref)
    h2 = norm_relu(gcn(h1, W2_ref, b2_ref), g2_ref, be2_ref)
    o_ref[...] = (jnp.dot(h2, lw_ref[...], preferred_element_type=jnp.float32)
                  + lb_ref[...])


def _tc_call(args, interpret=False):
    return pl.pallas_call(
        _tc_body,
        out_shape=jax.ShapeDtypeStruct((_N, _OUT), jnp.float32),
        interpret=interpret,
    )(*args)


def kernel(x, edge_index, edge_weight, weight, bias, attn_w, W1, b1, W2, b2,
           bn1_g, bn1_b, bn2_g, bn2_b, lin_W, lin_b):
    ei = jnp.asarray(edge_index, jnp.int32)
    zeros = jnp.zeros((_N * _N,), jnp.float32)
    acc = _sc_edge_scatter()(zeros, ei[0], ei[1],
                             jnp.asarray(edge_weight, jnp.float32))
    acc = acc.reshape(_N, _N)
    return _tc_call((
        x, weight,
        bias.reshape(1, -1), attn_w.reshape(1, -1),
        W1, b1.reshape(1, -1), W2, b2.reshape(1, -1),
        bn1_g.reshape(-1, 1), bn1_b.reshape(-1, 1),
        bn2_g.reshape(-1, 1), bn2_b.reshape(-1, 1),
        lin_W, lin_b.reshape(1, -1), acc,
    ))
